# Initial kernel scaffold; baseline (speedup 1.0000x reference)
import jax
import jax.numpy as jnp
from jax.experimental import pallas as pl


def kernel(x, edge_index, W1, b1, W2, b2, A1, A2, W3, b3):
    raise NotImplementedError("write your pallas kernel here")


# speedup vs baseline: 1.1421x; 1.1421x over previous
import functools

import jax
import jax.numpy as jnp
from jax.experimental import pallas as pl
from jax.experimental.pallas import tpu as pltpu

N = 10000
NH = 128
HOP = 10


def _mlp_body(x_ref, w1_ref, b1_ref, w2_ref, b2_ref, o_ref):
    h = jnp.maximum(
        jnp.dot(x_ref[...], w1_ref[...], preferred_element_type=jnp.float32)
        + b1_ref[...],
        0.0,
    )
    o_ref[...] = (
        jnp.dot(h, w2_ref[...], preferred_element_type=jnp.float32) + b2_ref[...]
    )


def _mlp(x, W1, b1, W2, b2, block=2000):
    n = x.shape[0]
    grid = (n // block,)
    return pl.pallas_call(
        _mlp_body,
        grid=grid,
        in_specs=[
            pl.BlockSpec((block, x.shape[1]), lambda i: (i, 0)),
            pl.BlockSpec((x.shape[1], W1.shape[1]), lambda i: (0, 0)),
            pl.BlockSpec((1, W1.shape[1]), lambda i: (0, 0)),
            pl.BlockSpec((W1.shape[1], W2.shape[1]), lambda i: (0, 0)),
            pl.BlockSpec((1, W2.shape[1]), lambda i: (0, 0)),
        ],
        out_specs=pl.BlockSpec((block, W2.shape[1]), lambda i: (i, 0)),
        out_shape=jax.ShapeDtypeStruct((n, W2.shape[1]), jnp.float32),
    )(x, W1, b1[None, :], W2, b2[None, :])


def _matmul_bias_body(x_ref, w_ref, b_ref, o_ref):
    o_ref[...] = (
        jnp.dot(x_ref[...], w_ref[...], preferred_element_type=jnp.float32)
        + b_ref[...]
    )


def _matmul_bias(x, W, b, block=2000):
    n = x.shape[0]
    return pl.pallas_call(
        _matmul_bias_body,
        grid=(n // block,),
        in_specs=[
            pl.BlockSpec((block, x.shape[1]), lambda i: (i, 0)),
            pl.BlockSpec((x.shape[1], W.shape[1]), lambda i: (0, 0)),
            pl.BlockSpec((1, W.shape[1]), lambda i: (0, 0)),
        ],
        out_specs=pl.BlockSpec((block, W.shape[1]), lambda i: (i, 0)),
        out_shape=jax.ShapeDtypeStruct((n, W.shape[1]), jnp.float32),
    )(x, W, b[None, :])


def kernel(x, edge_index, W1, b1, W2, b2, A1, A2, W3, b3):
    s = edge_index[0]
    t = edge_index[1]
    x = _mlp(x, W1, b1, W2, b2)
    h = x
    for i in range(HOP):
        a1 = A1[i]
        a2 = A2[i]
        x1 = (x @ a1)[:, None]
        h1 = (h @ a2)[:, None]
        w1 = x1[s] + h1[t]
        w2 = x1 + (x @ a2)[:, None]
        w1 = jnp.exp(jax.nn.leaky_relu(w1, 0.2))
        w2 = jnp.exp(jax.nn.leaky_relu(w2, 0.2))
        div = jax.ops.segment_sum(w1, s, num_segments=N) + w2
        h = jax.ops.segment_sum(w1 * h[t], s, num_segments=N) + w2 * x
        h = h / div
        h = jax.nn.elu(h)
    h = _matmul_bias(h, W3, b3)
    return h


# speedup vs baseline: 8.4566x; 7.4045x over previous
import functools

import jax
import jax.numpy as jnp
from jax import lax
from jax.experimental import pallas as pl
from jax.experimental.pallas import tpu as pltpu
from jax.experimental.pallas import tpu_sc as plsc

N = 10000
E = 320000
NH = 128
HOP = 10

NW = 32
RPT = 320
NPAD = NW * RPT
TRASH = RPT
CAP = 16384
CLAMP = CAP - 680
B = 64
CHK = 8000
NCHUNK = E // CHK
ACCW = 144


def _dot16(a, b_ref, off):
    return a * b_ref[pl.ds(off, 16)]


def _pre_body(x_ref, w1_ref, b1_ref, w2_ref, b2_ref, a1_ref, a2_ref, a20_ref,
              xm_ref, x1t_ref, w2t_ref, h10_ref):
    h = jnp.maximum(
        jnp.dot(x_ref[...], w1_ref[...], preferred_element_type=jnp.float32)
        + b1_ref[...], 0.0)
    xm = jnp.dot(h, w2_ref[...], preferred_element_type=jnp.float32) + b2_ref[...]
    xm_ref[...] = xm
    dn = (((1,), (1,)), ((), ()))
    x1t = lax.dot_general(a1_ref[...], xm, dn, preferred_element_type=jnp.float32)
    xa2t = lax.dot_general(a2_ref[...], xm, dn, preferred_element_type=jnp.float32)
    x1t_ref[...] = x1t
    pre = x1t + xa2t
    w2t_ref[...] = jnp.exp(jnp.where(pre >= 0, pre, 0.2 * pre))
    h10_ref[...] = lax.dot_general(xm, a20_ref[...], dn,
                                   preferred_element_type=jnp.float32)


def _preamble(xpad, W1, b1, W2, b2, A1p, A2p, a20, block=2048):
    grid = (NPAD // block,)
    return pl.pallas_call(
        _pre_body,
        grid=grid,
        in_specs=[
            pl.BlockSpec((block, NH), lambda i: (i, 0)),
            pl.BlockSpec((NH, NH), lambda i: (0, 0)),
            pl.BlockSpec((1, NH), lambda i: (0, 0)),
            pl.BlockSpec((NH, NH), lambda i: (0, 0)),
            pl.BlockSpec((1, NH), lambda i: (0, 0)),
            pl.BlockSpec((16, NH), lambda i: (0, 0)),
            pl.BlockSpec((16, NH), lambda i: (0, 0)),
            pl.BlockSpec((1, NH), lambda i: (0, 0)),
        ],
        out_specs=[
            pl.BlockSpec((block, NH), lambda i: (i, 0)),
            pl.BlockSpec((16, block), lambda i: (0, i)),
            pl.BlockSpec((16, block), lambda i: (0, i)),
            pl.BlockSpec((block, 1), lambda i: (i, 0)),
        ],
        out_shape=[
            jax.ShapeDtypeStruct((NPAD, NH), jnp.float32),
            jax.ShapeDtypeStruct((16, NPAD), jnp.float32),
            jax.ShapeDtypeStruct((16, NPAD), jnp.float32),
            jax.ShapeDtypeStruct((NPAD, 1), jnp.float32),
        ],
    )(xpad, W1, b1[None, :], W2, b2[None, :], A1p, A2p, a20)


def _post_body(x_ref, w_ref, b_ref, o_ref):
    o_ref[...] = (
        jnp.dot(x_ref[...], w_ref[...], preferred_element_type=jnp.float32)
        + b_ref[...])


def _postamble(h, W3, b3, block=2000):
    return pl.pallas_call(
        _post_body,
        grid=(N // block,),
        in_specs=[
            pl.BlockSpec((block, NH), lambda i: (i, 0)),
            pl.BlockSpec((NH, W3.shape[1]), lambda i: (0, 0)),
            pl.BlockSpec((1, W3.shape[1]), lambda i: (0, 0)),
        ],
        out_specs=pl.BlockSpec((block, W3.shape[1]), lambda i: (i, 0)),
        out_shape=jax.ShapeDtypeStruct((N, W3.shape[1]), jnp.float32),
    )(h, W3, b3[None, :])


_MESH = plsc.VectorSubcoreMesh(core_axis_name="c", subcore_axis_name="s")

_TRASH_PACKED = TRASH << 16


def _bucketize_body(s_hbm, t_hbm, lists_hbm, counts_hbm,
                    s0, t0, s1, t1, listbuf, cntv, sem0, sem1):
    wid = lax.axis_index("s") * 2 + lax.axis_index("c")
    base = wid * RPT

    trash = jnp.full((16,), _TRASH_PACKED, jnp.int32)
    def init_body(i, carry):
        listbuf[pl.ds(i * 16, 16)] = trash
        return carry
    lax.fori_loop(0, CAP // 16, init_body, 0)

    def start(c, sb, tb, sem):
        pltpu.make_async_copy(s_hbm.at[pl.ds(c * CHK, CHK)], sb, sem).start()
        pltpu.make_async_copy(t_hbm.at[pl.ds(c * CHK, CHK)], tb, sem).start()

    def wait(c, sb, tb, sem):
        pltpu.make_async_copy(s_hbm.at[pl.ds(c * CHK, CHK)], sb, sem).wait()
        pltpu.make_async_copy(t_hbm.at[pl.ds(c * CHK, CHK)], tb, sem).wait()

    def scan_chunk(sb, tb, cnt):
        def body(j, cnt):
            sv = sb[pl.ds(j * 16, 16)]
            tv = tb[pl.ds(j * 16, 16)]
            sl = sv - base
            msk = (sl >= 0) & (sl < RPT) & (cnt < CLAMP)
            packed = (sl << 16) | tv
            plsc.store_compressed(listbuf.at[pl.ds(cnt, 16)], packed, mask=msk)
            pc = plsc.all_reduce_population_count(msk)
            return cnt + pc[0]
        return lax.fori_loop(0, CHK // 16, body, cnt)

    start(0, s0, t0, sem0)
    cnt = jnp.int32(0)
    for c in range(NCHUNK):
        if c % 2 == 0:
            if c + 1 < NCHUNK:
                start(c + 1, s1, t1, sem1)
            wait(c, s0, t0, sem0)
            cnt = scan_chunk(s0, t0, cnt)
        else:
            if c + 1 < NCHUNK:
                start(c + 1, s0, t0, sem0)
            wait(c, s1, t1, sem1)
            cnt = scan_chunk(s1, t1, cnt)

    mp = (cnt + 2 * B - 1) & ~(2 * B - 1)
    pltpu.sync_copy(listbuf, lists_hbm.at[wid])
    cntv[...] = jnp.broadcast_to(mp, (16,))
    pltpu.sync_copy(cntv, counts_hbm.at[wid])


def _bucketize(s, t):
    kern = pl.kernel(
        _bucketize_body,
        out_type=[
            jax.ShapeDtypeStruct((NW, CAP), jnp.int32),
            jax.ShapeDtypeStruct((NW, 16), jnp.int32),
        ],
        mesh=_MESH,
        scratch_types=[
            pltpu.VMEM((CHK,), jnp.int32),
            pltpu.VMEM((CHK,), jnp.int32),
            pltpu.VMEM((CHK,), jnp.int32),
            pltpu.VMEM((CHK,), jnp.int32),
            pltpu.VMEM((CAP,), jnp.int32),
            pltpu.VMEM((16,), jnp.int32),
            pltpu.SemaphoreType.DMA,
            pltpu.SemaphoreType.DMA,
        ],
        compiler_params=pltpu.CompilerParams(needs_layout_passes=False),
    )
    return kern(s, t)


def _hop_body(h_hbm, xm_hbm, h1_hbm, x1_hbm, w2_hbm, a2n_hbm, lists_hbm,
              counts_hbm, hout_hbm, h1out_hbm,
              lb0, lb1, h1buf, x1own, w2own, a2nbuf, acc,
              stage0, stage1, tb0, tb1, sl0, sl1, w10, w11,
              cbuf, xmbuf, houtbuf, partial, h1outbuf,
              sem0, sem1, seml0, seml1):
    wid = lax.axis_index("s") * 2 + lax.axis_index("c")
    base = wid * RPT

    pltpu.sync_copy(h1_hbm, h1buf)
    pltpu.sync_copy(x1_hbm.at[pl.ds(base, RPT)], x1own.at[pl.ds(0, RPT)])
    pltpu.sync_copy(w2_hbm.at[pl.ds(base, RPT)], w2own)
    pltpu.sync_copy(a2n_hbm, a2nbuf)
    pltpu.sync_copy(counts_hbm.at[wid], cbuf)
    zero16 = jnp.zeros((16,), jnp.float32)
    x1own[pl.ds(RPT, 16)] = zero16

    def zero_body(i, carry):
        for c in range(ACCW // 16):
            acc[i, pl.ds(c * 16, 16)] = zero16
        return carry
    lax.fori_loop(0, RPT + 1, zero_body, 0)

    mp = cbuf[...][0]
    nbh = mp // (2 * B)

    lane0 = (lax.iota(jnp.int32, 16) == 0).astype(jnp.float32)

    def start_lchunk(b, lbuf, seml):
        pltpu.make_async_copy(lists_hbm.at[wid, pl.ds(b * B, B)], lbuf,
                              seml).start()

    def wait_lchunk(b, lbuf, seml):
        pltpu.make_async_copy(lists_hbm.at[wid, pl.ds(b * B, B)], lbuf,
                              seml).wait()

    def build(lbuf, tb, slb, w1b):
        for j in range(B // 16):
            pv = lbuf[pl.ds(j * 16, 16)]
            tv = pv & 0xFFFF
            sv = lax.shift_right_logical(pv, 16)
            tb[pl.ds(j * 16, 16)] = tv
            slb[pl.ds(j * 16, 16)] = sv
            pre = (plsc.load_gather(x1own, [sv])
                   + plsc.load_gather(h1buf, [tv]))
            w1b[pl.ds(j * 16, 16)] = jnp.exp(jnp.where(pre >= 0, pre, 0.2 * pre))

    def start_gather(tb, stage, sem):
        pltpu.make_async_copy(h_hbm.at[tb], stage, sem).start()

    def wait_gather(tb, stage, sem):
        pltpu.make_async_copy(h_hbm.at[tb], stage, sem).wait()

    def process(stage, slb, w1b):
        def body(g, carry):
            wv = w1b[pl.ds(g * 16, 16)]
            sv = slb[pl.ds(g * 16, 16)]
            for lane in range(16):
                e = g * 16 + lane
                w = wv[lane]
                srow = sv[lane]
                for c in range(8):
                    plsc.addupdate(acc.at[srow, pl.ds(c * 16, 16)],
                                   w * stage[e, pl.ds(c * 16, 16)])
                plsc.addupdate(acc.at[srow, pl.ds(128, 16)], w * lane0)
            return carry
        lax.fori_loop(0, B // 16, body, 0)

    pltpu.sync_copy(lists_hbm.at[wid, pl.ds(0, B)], lb0)
    build(lb0, tb0, sl0, w10)
    start_gather(tb0, stage0, sem0)
    start_lchunk(jnp.int32(1), lb1, seml1)

    def pair_body(i, carry):
        b0 = 2 * i
        wait_lchunk(b0 + 1, lb1, seml1)
        build(lb1, tb1, sl1, w11)
        start_gather(tb1, stage1, sem1)
        start_lchunk(b0 + 2, lb0, seml0)
        wait_gather(tb0, stage0, sem0)
        process(stage0, sl0, w10)
        wait_lchunk(b0 + 2, lb0, seml0)
        build(lb0, tb0, sl0, w10)
        start_gather(tb0, stage0, sem0)
        start_lchunk(b0 + 3, lb1, seml1)
        wait_gather(tb1, stage1, sem1)
        process(stage1, sl1, w11)
        return carry
    lax.fori_loop(0, nbh, pair_body, 0)
    wait_gather(tb0, stage0, sem0)
    wait_lchunk(jnp.int32(1), lb1, seml1)

    iota16 = lax.iota(jnp.int32, 16)
    c128 = jnp.full((16,), 128, jnp.int32)

    def grp_body(rg, carry):
        nl0 = rg * 16
        pltpu.sync_copy(xm_hbm.at[pl.ds(base + nl0, 16)], xmbuf)
        nlv = iota16 + nl0
        w2v = w2own[pl.ds(nl0, 16)]
        dvv = plsc.load_gather(acc, [nlv, c128]) + w2v
        rinv = 1.0 / dvv
        for lane in range(16):
            nl = nl0 + lane
            w2s = w2v[lane]
            rin = rinv[lane]
            dacc = jnp.zeros((16,), jnp.float32)
            for c in range(8):
                hv = (acc[nl, pl.ds(c * 16, 16)]
                      + w2s * xmbuf[lane, pl.ds(c * 16, 16)]) * rin
                hv = jnp.where(hv > 0, hv, jnp.exp(hv) - 1.0)
                houtbuf[lane, pl.ds(c * 16, 16)] = hv
                dacc = dacc + hv * a2nbuf[pl.ds(c * 16, 16)]
            partial[pl.ds(lane * 16, 16)] = dacc
        h1v = jnp.zeros((16,), jnp.float32)
        idxb = iota16 * 16
        for k in range(16):
            h1v = h1v + plsc.load_gather(partial, [idxb + k])
        h1outbuf[pl.ds(nl0, 16)] = h1v
        pltpu.sync_copy(houtbuf, hout_hbm.at[pl.ds(base + nl0, 16)])
        return carry
    lax.fori_loop(0, RPT // 16, grp_body, 0)
    pltpu.sync_copy(h1outbuf, h1out_hbm.at[pl.ds(base, RPT)])


def _hop(h, xm, h1, x1, w2, a2n, lists, counts):
    kern = pl.kernel(
        _hop_body,
        out_type=[
            jax.ShapeDtypeStruct((NPAD, NH), jnp.float32),
            jax.ShapeDtypeStruct((NPAD,), jnp.float32),
        ],
        mesh=_MESH,
        scratch_types=[
            pltpu.VMEM((B,), jnp.int32),
            pltpu.VMEM((B,), jnp.int32),
            pltpu.VMEM((NPAD,), jnp.float32),
            pltpu.VMEM((RPT + 16,), jnp.float32),
            pltpu.VMEM((RPT,), jnp.float32),
            pltpu.VMEM((NH,), jnp.float32),
            pltpu.VMEM((RPT + 1, ACCW), jnp.float32),
            pltpu.VMEM((B, NH), jnp.float32),
            pltpu.VMEM((B, NH), jnp.float32),
            pltpu.VMEM((B,), jnp.int32),
            pltpu.VMEM((B,), jnp.int32),
            pltpu.VMEM((B,), jnp.int32),
            pltpu.VMEM((B,), jnp.int32),
            pltpu.VMEM((B,), jnp.float32),
            pltpu.VMEM((B,), jnp.float32),
            pltpu.VMEM((16,), jnp.int32),
            pltpu.VMEM((16, NH), jnp.float32),
            pltpu.VMEM((16, NH), jnp.float32),
            pltpu.VMEM((256,), jnp.float32),
            pltpu.VMEM((RPT,), jnp.float32),
            pltpu.SemaphoreType.DMA,
            pltpu.SemaphoreType.DMA,
            pltpu.SemaphoreType.DMA,
            pltpu.SemaphoreType.DMA,
        ],
        compiler_params=pltpu.CompilerParams(needs_layout_passes=False),
    )
    return kern(h, xm, h1, x1, w2, a2n, lists, counts)


def kernel(x, edge_index, W1, b1, W2, b2, A1, A2, W3, b3):
    s = edge_index[0]
    t = edge_index[1]
    xpad = jnp.pad(x, ((0, NPAD - N), (0, 0)))
    A1p = jnp.pad(A1, ((0, 16 - HOP), (0, 0)))
    A2p = jnp.pad(A2, ((0, 16 - HOP), (0, 0)))
    xm, X1T, W2T, H10 = _preamble(xpad, W1, b1, W2, b2, A1p, A2p, A2[0:1])
    lists, counts = _bucketize(s, t)
    h = xm
    h1 = H10[:, 0]
    for i in range(HOP):
        h, h1 = _hop(h, xm, h1, X1T[i], W2T[i], A2[(i + 1) % HOP], lists, counts)
    return _postamble(h[:N], W3, b3)


# speedup vs baseline: 13.6121x; 1.6096x over previous
import functools

import jax
import jax.numpy as jnp
from jax import lax
from jax.experimental import pallas as pl
from jax.experimental.pallas import tpu as pltpu
from jax.experimental.pallas import tpu_sc as plsc

N = 10000
E = 320000
NH = 128
HOP = 10

NW = 32
RPT = 320
NPAD = NW * RPT
TRASH = RPT
CAP = 16384
CLAMP = CAP - 680
B = 64
CHK = 8000
NCHUNK = E // CHK
ACCW = 144


def _dot16(a, b_ref, off):
    return a * b_ref[pl.ds(off, 16)]


def _pre_body(x_ref, w1_ref, b1_ref, w2_ref, b2_ref, a1_ref, a2_ref, a20_ref,
              xm_ref, x1t_ref, w2t_ref, h10_ref):
    h = jnp.maximum(
        jnp.dot(x_ref[...], w1_ref[...], preferred_element_type=jnp.float32)
        + b1_ref[...], 0.0)
    xm = jnp.dot(h, w2_ref[...], preferred_element_type=jnp.float32) + b2_ref[...]
    xm_ref[...] = xm
    dn = (((1,), (1,)), ((), ()))
    x1t = lax.dot_general(a1_ref[...], xm, dn, preferred_element_type=jnp.float32)
    xa2t = lax.dot_general(a2_ref[...], xm, dn, preferred_element_type=jnp.float32)
    x1t_ref[...] = x1t
    pre = x1t + xa2t
    w2t_ref[...] = jnp.exp(jnp.where(pre >= 0, pre, 0.2 * pre))
    h10_ref[...] = lax.dot_general(xm, a20_ref[...], dn,
                                   preferred_element_type=jnp.float32)


def _preamble(xpad, W1, b1, W2, b2, A1p, A2p, a20, block=2048):
    grid = (NPAD // block,)
    return pl.pallas_call(
        _pre_body,
        grid=grid,
        in_specs=[
            pl.BlockSpec((block, NH), lambda i: (i, 0)),
            pl.BlockSpec((NH, NH), lambda i: (0, 0)),
            pl.BlockSpec((1, NH), lambda i: (0, 0)),
            pl.BlockSpec((NH, NH), lambda i: (0, 0)),
            pl.BlockSpec((1, NH), lambda i: (0, 0)),
            pl.BlockSpec((16, NH), lambda i: (0, 0)),
            pl.BlockSpec((16, NH), lambda i: (0, 0)),
            pl.BlockSpec((1, NH), lambda i: (0, 0)),
        ],
        out_specs=[
            pl.BlockSpec((block, NH), lambda i: (i, 0)),
            pl.BlockSpec((16, block), lambda i: (0, i)),
            pl.BlockSpec((16, block), lambda i: (0, i)),
            pl.BlockSpec((block, 1), lambda i: (i, 0)),
        ],
        out_shape=[
            jax.ShapeDtypeStruct((NPAD, NH), jnp.float32),
            jax.ShapeDtypeStruct((16, NPAD), jnp.float32),
            jax.ShapeDtypeStruct((16, NPAD), jnp.float32),
            jax.ShapeDtypeStruct((NPAD, 1), jnp.float32),
        ],
    )(xpad, W1, b1[None, :], W2, b2[None, :], A1p, A2p, a20)


def _post_body(x_ref, w_ref, b_ref, o_ref):
    o_ref[...] = (
        jnp.dot(x_ref[...], w_ref[...], preferred_element_type=jnp.float32)
        + b_ref[...])


def _postamble(h, W3, b3, block=2000):
    return pl.pallas_call(
        _post_body,
        grid=(N // block,),
        in_specs=[
            pl.BlockSpec((block, NH), lambda i: (i, 0)),
            pl.BlockSpec((NH, W3.shape[1]), lambda i: (0, 0)),
            pl.BlockSpec((1, W3.shape[1]), lambda i: (0, 0)),
        ],
        out_specs=pl.BlockSpec((block, W3.shape[1]), lambda i: (i, 0)),
        out_shape=jax.ShapeDtypeStruct((N, W3.shape[1]), jnp.float32),
    )(h, W3, b3[None, :])


_MESH = plsc.VectorSubcoreMesh(core_axis_name="c", subcore_axis_name="s")

_TRASH_PACKED = TRASH << 16


def _bucketize_body(s_hbm, t_hbm, lists_hbm, counts_hbm,
                    s0, t0, s1, t1, listbuf, cntv, sem0, sem1):
    wid = lax.axis_index("s") * 2 + lax.axis_index("c")
    base = wid * RPT

    trash = jnp.full((16,), _TRASH_PACKED, jnp.int32)
    def init_body(i, carry):
        listbuf[pl.ds(i * 16, 16)] = trash
        return carry
    lax.fori_loop(0, CAP // 16, init_body, 0)

    def start(c, sb, tb, sem):
        pltpu.make_async_copy(s_hbm.at[pl.ds(c * CHK, CHK)], sb, sem).start()
        pltpu.make_async_copy(t_hbm.at[pl.ds(c * CHK, CHK)], tb, sem).start()

    def wait(c, sb, tb, sem):
        pltpu.make_async_copy(s_hbm.at[pl.ds(c * CHK, CHK)], sb, sem).wait()
        pltpu.make_async_copy(t_hbm.at[pl.ds(c * CHK, CHK)], tb, sem).wait()

    def scan_chunk(sb, tb, cnt):
        def body(j, cnt):
            sv = sb[pl.ds(j * 16, 16)]
            tv = tb[pl.ds(j * 16, 16)]
            sl = sv - base
            msk = (sl >= 0) & (sl < RPT) & (cnt < CLAMP)
            packed = (sl << 16) | tv
            plsc.store_compressed(listbuf.at[pl.ds(cnt, 16)], packed, mask=msk)
            pc = plsc.all_reduce_population_count(msk)
            return cnt + pc[0]
        return lax.fori_loop(0, CHK // 16, body, cnt)

    start(0, s0, t0, sem0)
    cnt = jnp.int32(0)
    for c in range(NCHUNK):
        if c % 2 == 0:
            if c + 1 < NCHUNK:
                start(c + 1, s1, t1, sem1)
            wait(c, s0, t0, sem0)
            cnt = scan_chunk(s0, t0, cnt)
        else:
            if c + 1 < NCHUNK:
                start(c + 1, s0, t0, sem0)
            wait(c, s1, t1, sem1)
            cnt = scan_chunk(s1, t1, cnt)

    mp = (cnt + 2 * B - 1) & ~(2 * B - 1)
    pltpu.sync_copy(listbuf, lists_hbm.at[wid])
    cntv[...] = jnp.broadcast_to(mp, (16,))
    pltpu.sync_copy(cntv, counts_hbm.at[wid])


def _bucketize(s, t):
    kern = pl.kernel(
        _bucketize_body,
        out_type=[
            jax.ShapeDtypeStruct((NW, CAP), jnp.int32),
            jax.ShapeDtypeStruct((NW, 16), jnp.int32),
        ],
        mesh=_MESH,
        scratch_types=[
            pltpu.VMEM((CHK,), jnp.int32),
            pltpu.VMEM((CHK,), jnp.int32),
            pltpu.VMEM((CHK,), jnp.int32),
            pltpu.VMEM((CHK,), jnp.int32),
            pltpu.VMEM((CAP,), jnp.int32),
            pltpu.VMEM((16,), jnp.int32),
            pltpu.SemaphoreType.DMA,
            pltpu.SemaphoreType.DMA,
        ],
        compiler_params=pltpu.CompilerParams(needs_layout_passes=False),
    )
    return kern(s, t)


def _hop_body(h_hbm, xm_hbm, h1_hbm, x1_hbm, w2_hbm, a2n_hbm, lists_hbm,
              counts_hbm, hout_hbm, h1out_hbm,
              lb0, lb1, h1buf, x1own, w2own, a2nbuf, acc,
              stage0, stage1, tb0, tb1, sl0, sl1, w10, w11,
              cbuf, xmbuf, houtbuf, partial, h1outbuf,
              sem0, sem1, seml0, seml1):
    wid = lax.axis_index("s") * 2 + lax.axis_index("c")
    base = wid * RPT

    pltpu.sync_copy(h1_hbm, h1buf)
    pltpu.sync_copy(x1_hbm.at[pl.ds(base, RPT)], x1own.at[pl.ds(0, RPT)])
    pltpu.sync_copy(w2_hbm.at[pl.ds(base, RPT)], w2own)
    pltpu.sync_copy(a2n_hbm, a2nbuf)
    pltpu.sync_copy(counts_hbm.at[wid], cbuf)
    zero16 = jnp.zeros((16,), jnp.float32)
    x1own[pl.ds(RPT, 16)] = zero16

    def zero_body(i, carry):
        for c in range(ACCW // 16):
            acc[i, pl.ds(c * 16, 16)] = zero16
        return carry
    lax.fori_loop(0, RPT + 1, zero_body, 0)

    mp = cbuf[...][0]
    nbh = mp // (2 * B)

    lane0 = (lax.iota(jnp.int32, 16) == 0).astype(jnp.float32)

    def start_lchunk(b, lbuf, seml):
        pltpu.make_async_copy(lists_hbm.at[wid, pl.ds(b * B, B)], lbuf,
                              seml).start()

    def wait_lchunk(b, lbuf, seml):
        pltpu.make_async_copy(lists_hbm.at[wid, pl.ds(b * B, B)], lbuf,
                              seml).wait()

    def build(lbuf, tb, slb, w1b):
        for j in range(B // 16):
            pv = lbuf[pl.ds(j * 16, 16)]
            tv = pv & 0xFFFF
            sv = lax.shift_right_logical(pv, 16)
            tb[pl.ds(j * 16, 16)] = tv
            pre = (plsc.load_gather(x1own, [sv])
                   + plsc.load_gather(h1buf, [tv]))
            w1v = jnp.exp(jnp.where(pre >= 0, pre, 0.2 * pre))
            for lane in range(16):
                slb[j * 16 + lane] = sv[lane]
                w1b[j * 16 + lane] = w1v[lane]

    def start_gather(tb, stage, sem):
        pltpu.make_async_copy(h_hbm.at[tb], stage, sem).start()

    def wait_gather(tb, stage, sem):
        pltpu.make_async_copy(h_hbm.at[tb], stage, sem).wait()

    def process(stage, slb, w1b):
        @plsc.parallel_loop(0, B, unroll=4)
        def body(e):
            w = w1b[e]
            srow = slb[e]
            for c in range(8):
                plsc.addupdate(acc.at[srow, pl.ds(c * 16, 16)],
                               w * stage[e, pl.ds(c * 16, 16)])
            plsc.addupdate(acc.at[srow, pl.ds(128, 16)], w * lane0)

    pltpu.sync_copy(lists_hbm.at[wid, pl.ds(0, B)], lb0)
    build(lb0, tb0, sl0, w10)
    start_gather(tb0, stage0, sem0)
    start_lchunk(jnp.int32(1), lb1, seml1)

    def pair_body(i, carry):
        b0 = 2 * i
        wait_lchunk(b0 + 1, lb1, seml1)
        build(lb1, tb1, sl1, w11)
        start_gather(tb1, stage1, sem1)
        start_lchunk(b0 + 2, lb0, seml0)
        wait_gather(tb0, stage0, sem0)
        process(stage0, sl0, w10)
        wait_lchunk(b0 + 2, lb0, seml0)
        build(lb0, tb0, sl0, w10)
        start_gather(tb0, stage0, sem0)
        start_lchunk(b0 + 3, lb1, seml1)
        wait_gather(tb1, stage1, sem1)
        process(stage1, sl1, w11)
        return carry
    lax.fori_loop(0, nbh, pair_body, 0)
    wait_gather(tb0, stage0, sem0)
    wait_lchunk(jnp.int32(1), lb1, seml1)

    iota16 = lax.iota(jnp.int32, 16)
    c128 = jnp.full((16,), 128, jnp.int32)

    def grp_body(rg, carry):
        nl0 = rg * 16
        pltpu.sync_copy(xm_hbm.at[pl.ds(base + nl0, 16)], xmbuf)
        nlv = iota16 + nl0
        w2v = w2own[pl.ds(nl0, 16)]
        dvv = plsc.load_gather(acc, [nlv, c128]) + w2v
        rinv = 1.0 / dvv
        for lane in range(16):
            nl = nl0 + lane
            w2s = w2v[lane]
            rin = rinv[lane]
            dacc = jnp.zeros((16,), jnp.float32)
            for c in range(8):
                hv = (acc[nl, pl.ds(c * 16, 16)]
                      + w2s * xmbuf[lane, pl.ds(c * 16, 16)]) * rin
                hv = jnp.where(hv > 0, hv, jnp.exp(hv) - 1.0)
                houtbuf[lane, pl.ds(c * 16, 16)] = hv
                dacc = dacc + hv * a2nbuf[pl.ds(c * 16, 16)]
            partial[pl.ds(lane * 16, 16)] = dacc
        h1v = jnp.zeros((16,), jnp.float32)
        idxb = iota16 * 16
        for k in range(16):
            h1v = h1v + plsc.load_gather(partial, [idxb + k])
        h1outbuf[pl.ds(nl0, 16)] = h1v
        pltpu.sync_copy(houtbuf, hout_hbm.at[pl.ds(base + nl0, 16)])
        return carry
    lax.fori_loop(0, RPT // 16, grp_body, 0)
    pltpu.sync_copy(h1outbuf, h1out_hbm.at[pl.ds(base, RPT)])


def _hop(h, xm, h1, x1, w2, a2n, lists, counts):
    kern = pl.kernel(
        _hop_body,
        out_type=[
            jax.ShapeDtypeStruct((NPAD, NH), jnp.float32),
            jax.ShapeDtypeStruct((NPAD,), jnp.float32),
        ],
        mesh=_MESH,
        scratch_types=[
            pltpu.VMEM((B,), jnp.int32),
            pltpu.VMEM((B,), jnp.int32),
            pltpu.VMEM((NPAD,), jnp.float32),
            pltpu.VMEM((RPT + 16,), jnp.float32),
            pltpu.VMEM((RPT,), jnp.float32),
            pltpu.VMEM((NH,), jnp.float32),
            pltpu.VMEM((RPT + 1, ACCW), jnp.float32),
            pltpu.VMEM((B, NH), jnp.float32),
            pltpu.VMEM((B, NH), jnp.float32),
            pltpu.VMEM((B,), jnp.int32),
            pltpu.VMEM((B,), jnp.int32),
            pltpu.SMEM((B,), jnp.int32),
            pltpu.SMEM((B,), jnp.int32),
            pltpu.SMEM((B,), jnp.float32),
            pltpu.SMEM((B,), jnp.float32),
            pltpu.VMEM((16,), jnp.int32),
            pltpu.VMEM((16, NH), jnp.float32),
            pltpu.VMEM((16, NH), jnp.float32),
            pltpu.VMEM((256,), jnp.float32),
            pltpu.VMEM((RPT,), jnp.float32),
            pltpu.SemaphoreType.DMA,
            pltpu.SemaphoreType.DMA,
            pltpu.SemaphoreType.DMA,
            pltpu.SemaphoreType.DMA,
        ],
        compiler_params=pltpu.CompilerParams(needs_layout_passes=False),
    )
    return kern(h, xm, h1, x1, w2, a2n, lists, counts)


def kernel(x, edge_index, W1, b1, W2, b2, A1, A2, W3, b3):
    s = edge_index[0]
    t = edge_index[1]
    xpad = jnp.pad(x, ((0, NPAD - N), (0, 0)))
    A1p = jnp.pad(A1, ((0, 16 - HOP), (0, 0)))
    A2p = jnp.pad(A2, ((0, 16 - HOP), (0, 0)))
    xm, X1T, W2T, H10 = _preamble(xpad, W1, b1, W2, b2, A1p, A2p, A2[0:1])
    lists, counts = _bucketize(s, t)
    h = xm
    h1 = H10[:, 0]
    for i in range(HOP):
        h, h1 = _hop(h, xm, h1, X1T[i], W2T[i], A2[(i + 1) % HOP], lists, counts)
    return _postamble(h[:N], W3, b3)
